# baseline (speedup 1.0000x reference)
import functools

import jax
import jax.numpy as jnp
from jax import lax
from jax.experimental import pallas as pl
from jax.experimental.pallas import tpu as pltpu
from jax.experimental.pallas import tpu_sc as plsc

B = 4096
DENSE = 13
OTHER = 24
BEH = 2
L = 50
ED = 16
VOCAB_BEH = 1000000
VOCAB_OTHER = 100000

NC, NS = 2, 16
NW = NC * NS
GL = 128
CG = 16

BEH_G = B * (L + 1) // GL
BEH_C = BEH_G // CG
SP_G = B * OTHER // GL
SP_C = SP_G // CG
SP_CPT = SP_C // OTHER


def _sc_gather_kernel(beh_tab, sp_tab, b0_idx, b1_idx, sp_idx,
                      b0_out, b1_out, sp_out, idx_v, rows_v, sem):
    wid = lax.axis_index("s") * NC + lax.axis_index("c")

    def gather_chunk(table, idx_hbm, out_hbm, cid):
        pltpu.sync_copy(idx_hbm.at[cid], idx_v)
        cps = [pltpu.async_copy(table.at[idx_v.at[j]], rows_v.at[j], sem)
               for j in range(CG)]
        for cp in cps:
            cp.wait()
        pltpu.sync_copy(rows_v, out_hbm.at[pl.ds(cid * CG, CG)])

    def run_job(get_table, idx_hbm, out_hbm, nchunk):
        def chunk(c):
            cid = wid + c * NW

            @pl.when(cid < nchunk)
            def _():
                gather_chunk(get_table(cid), idx_hbm, out_hbm, cid)

        trips = (nchunk + NW - 1) // NW
        if trips == 1:
            chunk(0)
        else:
            lax.fori_loop(0, trips, lambda c, _: (chunk(c), 0)[1], 0,
                          unroll=False)

    run_job(lambda cid: beh_tab.at[0], b0_idx, b0_out, BEH_C)
    run_job(lambda cid: beh_tab.at[1], b1_idx, b1_out, BEH_C)
    run_job(lambda cid: sp_tab.at[cid // SP_CPT], sp_idx, sp_out, SP_C)


@jax.jit
def _sc_gather(beh_tab, sp_tab, b0_idx, b1_idx, sp_idx):
    mesh = plsc.VectorSubcoreMesh(core_axis_name="c", subcore_axis_name="s",
                                  num_cores=NC, num_subcores=NS)
    return pl.kernel(
        _sc_gather_kernel,
        out_type=(
            jax.ShapeDtypeStruct((BEH_G, GL, ED), jnp.float32),
            jax.ShapeDtypeStruct((BEH_G, GL, ED), jnp.float32),
            jax.ShapeDtypeStruct((SP_G, GL, ED), jnp.float32),
        ),
        mesh=mesh,
        scratch_types=[
            pltpu.VMEM((CG, GL), jnp.int32),
            pltpu.VMEM((CG, GL, ED), jnp.float32),
            pltpu.SemaphoreType.DMA,
        ],
        compiler_params=pltpu.CompilerParams(use_tc_tiling_on_sc=False),
    )(beh_tab, sp_tab, b0_idx, b1_idx, sp_idx)


def _tc_dense_kernel(s0_ref, s1_ref, i0_ref, i1_ref, seq0_ref, dense_ref,
                     sp_ref,
                     w1_ref, b1_ref, a1_ref, w2_ref, b2_ref, a2_ref,
                     wf_ref, bf_ref,
                     g_u, g_i, g_d, g_s, be_u, be_i, be_d, be_s,
                     mu_u, mu_i, mu_d, mu_s, va_u, va_i, va_d, va_s,
                     f1u_ref, f1i_ref, f1d_ref, f1s_ref, fb1_ref, fa1_ref,
                     f2_ref, fb2_ref, fa2_ref, ow_ref, ob_ref, out_ref):
    bb = i0_ref.shape[1]
    seq = jnp.concatenate([s0_ref[...], s1_ref[...]], axis=-1)
    item = jnp.concatenate([i0_ref[0], i1_ref[0]], axis=-1)

    w1 = w1_ref[...]
    wq = w1[0:32] + w1[64:96]
    wk = w1[32:64] - w1[64:96]
    wqk = w1[96:128]

    def prelu(x, a):
        return jnp.where(x >= 0, x, a * x)

    hq = jnp.dot(item, wq, preferred_element_type=jnp.float32) + b1_ref[...]
    sf = seq.reshape(L * bb, 32)
    xf = (item[None, :, :] * seq).reshape(L * bb, 32)
    h = (jnp.broadcast_to(hq[None], (L, bb, 80)).reshape(L * bb, 80)
         + jnp.dot(sf, wk, preferred_element_type=jnp.float32)
         + jnp.dot(xf, wqk, preferred_element_type=jnp.float32))
    h = prelu(h, a1_ref[...])
    h = prelu(jnp.dot(h, w2_ref[...], preferred_element_type=jnp.float32)
              + b2_ref[...], a2_ref[...])
    scores = (h.reshape(L, bb, 40) * wf_ref[...]).sum(axis=-1) + bf_ref[0, 0]

    neg = jnp.float32(-2.0 ** 32 + 1.0)
    scores = jnp.where(seq0_ref[...] == 0, neg, scores)
    m = jnp.max(scores, axis=0, keepdims=True)
    e = jnp.exp(scores - m)
    w = e / jnp.sum(e, axis=0, keepdims=True)

    user = (w[:, :, None] * seq).sum(axis=0)

    def bn(x, g, be, mu, va):
        return (x - mu[...]) * lax.rsqrt(va[...] + 1e-3) * g[...] + be[...]

    xu = bn(user, g_u, be_u, mu_u, va_u)
    xi = bn(item, g_i, be_i, mu_i, va_i)
    xd = bn(dense_ref[...], g_d, be_d, mu_d, va_d)
    xs = bn(sp_ref[...], g_s, be_s, mu_s, va_s)

    x = (jnp.dot(xu, f1u_ref[...], preferred_element_type=jnp.float32)
         + jnp.dot(xi, f1i_ref[...], preferred_element_type=jnp.float32)
         + jnp.dot(xd, f1d_ref[...], preferred_element_type=jnp.float32)
         + fb1_ref[...])
    for k in range(OTHER):
        x = x + jnp.dot(xs[k], f1s_ref[k],
                        preferred_element_type=jnp.float32)
    x = prelu(x, fa1_ref[...])
    x = prelu(jnp.dot(x, f2_ref[...], preferred_element_type=jnp.float32)
              + fb2_ref[...], fa2_ref[...])
    logit = (x * ow_ref[...]).sum(axis=-1, keepdims=True) + ob_ref[0, 0]
    out_ref[...] = 1.0 / (1.0 + jnp.exp(-logit))


def _tc_dense(bb, b0, b1, seq0, dense, sp, params):
    nblk = B // bb
    full = lambda shape: pl.BlockSpec(shape, lambda i, s=shape: (0,) * len(s))
    in_specs = [
        pl.BlockSpec((L, bb, ED), lambda i: (0, i, 0)),
        pl.BlockSpec((L, bb, ED), lambda i: (0, i, 0)),
        pl.BlockSpec((1, bb, ED), lambda i: (L, i, 0)),
        pl.BlockSpec((1, bb, ED), lambda i: (L, i, 0)),
        pl.BlockSpec((L, bb), lambda i: (0, i)),
        pl.BlockSpec((bb, DENSE), lambda i: (i, 0)),
        pl.BlockSpec((OTHER, bb, ED), lambda i: (0, i, 0)),
    ] + [full(p.shape) for p in params]
    return pl.pallas_call(
        _tc_dense_kernel,
        grid=(nblk,),
        in_specs=in_specs,
        out_specs=pl.BlockSpec((bb, 1), lambda i: (i, 0)),
        out_shape=jax.ShapeDtypeStruct((B, 1), jnp.float32),
        compiler_params=pltpu.CompilerParams(
            dimension_semantics=("arbitrary",)),
    )(b0, b1, b0, b1, seq0, dense, sp, *params)


def kernel(dense_inputs, sparse_inputs, seq_inputs, item_inputs,
           sparse_tables, behavior_tables, att_W1, att_b1, att_a1,
           att_W2, att_b2, att_a2, att_Wf, att_bf, bn_gamma, bn_beta,
           bn_mean, bn_var, ffn_W1, ffn_b1, ffn_a1, ffn_W2, ffn_b2,
           ffn_a2, out_W, out_b):
    b0_idx = jnp.concatenate(
        [seq_inputs[:, :, 0].T.reshape(-1), item_inputs[:, 0]]
    ).reshape(BEH_C, CG, GL)
    b1_idx = jnp.concatenate(
        [seq_inputs[:, :, 1].T.reshape(-1), item_inputs[:, 1]]
    ).reshape(BEH_C, CG, GL)
    sp_idx = sparse_inputs.T.reshape(SP_C, CG, GL)

    b0_rows, b1_rows, sp_rows = _sc_gather(
        behavior_tables, sparse_tables, b0_idx, b1_idx, sp_idx)
    b0 = b0_rows.reshape(L + 1, B, ED)
    b1 = b1_rows.reshape(L + 1, B, ED)
    sp_e = sp_rows.reshape(OTHER, B, ED)

    seq0 = seq_inputs[:, :, 0].T

    r1 = lambda v: v.reshape(1, -1)
    o_u, o_i, o_d = 0, 32, 64
    o_s, o_e = 64 + DENSE, 64 + DENSE + OTHER * ED
    sl = lambda v: (r1(v[o_u:o_i]), r1(v[o_i:o_d]), r1(v[o_d:o_s]),
                    v[o_s:o_e].reshape(OTHER, 1, ED))
    g4, be4, mu4, va4 = sl(bn_gamma), sl(bn_beta), sl(bn_mean), sl(bn_var)

    params = (att_W1, r1(att_b1), r1(att_a1), att_W2, r1(att_b2),
              r1(att_a2), att_Wf.reshape(1, 1, 40), r1(att_bf),
              *g4, *be4, *mu4, *va4,
              ffn_W1[o_u:o_i], ffn_W1[o_i:o_d], ffn_W1[o_d:o_s],
              ffn_W1[o_s:o_e].reshape(OTHER, ED, 80), r1(ffn_b1),
              r1(ffn_a1), ffn_W2, r1(ffn_b2), r1(ffn_a2),
              out_W.reshape(1, 40), r1(out_b))
    return _tc_dense(256, b0, b1, seq0, dense_inputs, sp_e, params)


# speedup vs baseline: 1.0031x; 1.0031x over previous
import functools

import jax
import jax.numpy as jnp
from jax import lax
from jax.experimental import pallas as pl
from jax.experimental.pallas import tpu as pltpu
from jax.experimental.pallas import tpu_sc as plsc

B = 4096
DENSE = 13
OTHER = 24
BEH = 2
L = 50
ED = 16
VOCAB_BEH = 1000000
VOCAB_OTHER = 100000

NC, NS = 2, 16
NW = NC * NS
GL = 128
CG = 32

BEH_G = B * (L + 1) // GL
BEH_C = BEH_G // CG
SP_G = B * OTHER // GL
SP_C = SP_G // CG
SP_CPT = SP_C // OTHER


def _sc_gather_kernel(beh_tab, sp_tab, b0_idx, b1_idx, sp_idx,
                      b0_out, b1_out, sp_out, idx_v, rows_v, sem):
    wid = lax.axis_index("s") * NC + lax.axis_index("c")

    def gather_chunk(table, idx_hbm, out_hbm, cid):
        pltpu.sync_copy(idx_hbm.at[cid], idx_v)
        cps = [pltpu.async_copy(table.at[idx_v.at[j]], rows_v.at[j], sem)
               for j in range(CG)]
        for cp in cps:
            cp.wait()
        pltpu.sync_copy(rows_v, out_hbm.at[pl.ds(cid * CG, CG)])

    def run_job(get_table, idx_hbm, out_hbm, nchunk):
        def chunk(c):
            cid = wid + c * NW

            @pl.when(cid < nchunk)
            def _():
                gather_chunk(get_table(cid), idx_hbm, out_hbm, cid)

        trips = (nchunk + NW - 1) // NW
        if trips == 1:
            chunk(0)
        else:
            lax.fori_loop(0, trips, lambda c, _: (chunk(c), 0)[1], 0,
                          unroll=False)

    run_job(lambda cid: beh_tab.at[0], b0_idx, b0_out, BEH_C)
    run_job(lambda cid: beh_tab.at[1], b1_idx, b1_out, BEH_C)
    run_job(lambda cid: sp_tab.at[cid // SP_CPT], sp_idx, sp_out, SP_C)


@jax.jit
def _sc_gather(beh_tab, sp_tab, b0_idx, b1_idx, sp_idx):
    mesh = plsc.VectorSubcoreMesh(core_axis_name="c", subcore_axis_name="s",
                                  num_cores=NC, num_subcores=NS)
    return pl.kernel(
        _sc_gather_kernel,
        out_type=(
            jax.ShapeDtypeStruct((BEH_G, GL, ED), jnp.float32),
            jax.ShapeDtypeStruct((BEH_G, GL, ED), jnp.float32),
            jax.ShapeDtypeStruct((SP_G, GL, ED), jnp.float32),
        ),
        mesh=mesh,
        scratch_types=[
            pltpu.VMEM((CG, GL), jnp.int32),
            pltpu.VMEM((CG, GL, ED), jnp.float32),
            pltpu.SemaphoreType.DMA,
        ],
        compiler_params=pltpu.CompilerParams(use_tc_tiling_on_sc=False),
    )(beh_tab, sp_tab, b0_idx, b1_idx, sp_idx)


def _tc_dense_kernel(s0_ref, s1_ref, i0_ref, i1_ref, seq0_ref, dense_ref,
                     sp_ref,
                     w1_ref, b1_ref, a1_ref, w2_ref, b2_ref, a2_ref,
                     wf_ref, bf_ref,
                     g_u, g_i, g_d, g_s, be_u, be_i, be_d, be_s,
                     mu_u, mu_i, mu_d, mu_s, va_u, va_i, va_d, va_s,
                     f1u_ref, f1i_ref, f1d_ref, f1s_ref, fb1_ref, fa1_ref,
                     f2_ref, fb2_ref, fa2_ref, ow_ref, ob_ref, out_ref):
    bb = i0_ref.shape[1]
    seq = jnp.concatenate([s0_ref[...], s1_ref[...]], axis=-1)
    item = jnp.concatenate([i0_ref[0], i1_ref[0]], axis=-1)

    w1 = w1_ref[...]
    wq = w1[0:32] + w1[64:96]
    wk = w1[32:64] - w1[64:96]
    wqk = w1[96:128]

    def prelu(x, a):
        return jnp.where(x >= 0, x, a * x)

    hq = jnp.dot(item, wq, preferred_element_type=jnp.float32) + b1_ref[...]
    sf = seq.reshape(L * bb, 32)
    xf = (item[None, :, :] * seq).reshape(L * bb, 32)
    h = (jnp.broadcast_to(hq[None], (L, bb, 80)).reshape(L * bb, 80)
         + jnp.dot(sf, wk, preferred_element_type=jnp.float32)
         + jnp.dot(xf, wqk, preferred_element_type=jnp.float32))
    h = prelu(h, a1_ref[...])
    h = prelu(jnp.dot(h, w2_ref[...], preferred_element_type=jnp.float32)
              + b2_ref[...], a2_ref[...])
    scores = (h.reshape(L, bb, 40) * wf_ref[...]).sum(axis=-1) + bf_ref[0, 0]

    neg = jnp.float32(-2.0 ** 32 + 1.0)
    scores = jnp.where(seq0_ref[...] == 0, neg, scores)
    m = jnp.max(scores, axis=0, keepdims=True)
    e = jnp.exp(scores - m)
    w = e / jnp.sum(e, axis=0, keepdims=True)

    user = (w[:, :, None] * seq).sum(axis=0)

    def bn(x, g, be, mu, va):
        return (x - mu[...]) * lax.rsqrt(va[...] + 1e-3) * g[...] + be[...]

    xu = bn(user, g_u, be_u, mu_u, va_u)
    xi = bn(item, g_i, be_i, mu_i, va_i)
    xd = bn(dense_ref[...], g_d, be_d, mu_d, va_d)
    xs = bn(sp_ref[...], g_s, be_s, mu_s, va_s)

    x = (jnp.dot(xu, f1u_ref[...], preferred_element_type=jnp.float32)
         + jnp.dot(xi, f1i_ref[...], preferred_element_type=jnp.float32)
         + jnp.dot(xd, f1d_ref[...], preferred_element_type=jnp.float32)
         + fb1_ref[...])
    for k in range(OTHER):
        x = x + jnp.dot(xs[k], f1s_ref[k],
                        preferred_element_type=jnp.float32)
    x = prelu(x, fa1_ref[...])
    x = prelu(jnp.dot(x, f2_ref[...], preferred_element_type=jnp.float32)
              + fb2_ref[...], fa2_ref[...])
    logit = (x * ow_ref[...]).sum(axis=-1, keepdims=True) + ob_ref[0, 0]
    out_ref[...] = 1.0 / (1.0 + jnp.exp(-logit))


def _tc_dense(bb, b0, b1, seq0, dense, sp, params):
    nblk = B // bb
    full = lambda shape: pl.BlockSpec(shape, lambda i, s=shape: (0,) * len(s))
    in_specs = [
        pl.BlockSpec((L, bb, ED), lambda i: (0, i, 0)),
        pl.BlockSpec((L, bb, ED), lambda i: (0, i, 0)),
        pl.BlockSpec((1, bb, ED), lambda i: (L, i, 0)),
        pl.BlockSpec((1, bb, ED), lambda i: (L, i, 0)),
        pl.BlockSpec((L, bb), lambda i: (0, i)),
        pl.BlockSpec((bb, DENSE), lambda i: (i, 0)),
        pl.BlockSpec((OTHER, bb, ED), lambda i: (0, i, 0)),
    ] + [full(p.shape) for p in params]
    return pl.pallas_call(
        _tc_dense_kernel,
        grid=(nblk,),
        in_specs=in_specs,
        out_specs=pl.BlockSpec((bb, 1), lambda i: (i, 0)),
        out_shape=jax.ShapeDtypeStruct((B, 1), jnp.float32),
        compiler_params=pltpu.CompilerParams(
            dimension_semantics=("arbitrary",)),
    )(b0, b1, b0, b1, seq0, dense, sp, *params)


def kernel(dense_inputs, sparse_inputs, seq_inputs, item_inputs,
           sparse_tables, behavior_tables, att_W1, att_b1, att_a1,
           att_W2, att_b2, att_a2, att_Wf, att_bf, bn_gamma, bn_beta,
           bn_mean, bn_var, ffn_W1, ffn_b1, ffn_a1, ffn_W2, ffn_b2,
           ffn_a2, out_W, out_b):
    b0_idx = jnp.concatenate(
        [seq_inputs[:, :, 0].T.reshape(-1), item_inputs[:, 0]]
    ).reshape(BEH_C, CG, GL)
    b1_idx = jnp.concatenate(
        [seq_inputs[:, :, 1].T.reshape(-1), item_inputs[:, 1]]
    ).reshape(BEH_C, CG, GL)
    sp_idx = sparse_inputs.T.reshape(SP_C, CG, GL)

    b0_rows, b1_rows, sp_rows = _sc_gather(
        behavior_tables, sparse_tables, b0_idx, b1_idx, sp_idx)
    b0 = b0_rows.reshape(L + 1, B, ED)
    b1 = b1_rows.reshape(L + 1, B, ED)
    sp_e = sp_rows.reshape(OTHER, B, ED)

    seq0 = seq_inputs[:, :, 0].T

    r1 = lambda v: v.reshape(1, -1)
    o_u, o_i, o_d = 0, 32, 64
    o_s, o_e = 64 + DENSE, 64 + DENSE + OTHER * ED
    sl = lambda v: (r1(v[o_u:o_i]), r1(v[o_i:o_d]), r1(v[o_d:o_s]),
                    v[o_s:o_e].reshape(OTHER, 1, ED))
    g4, be4, mu4, va4 = sl(bn_gamma), sl(bn_beta), sl(bn_mean), sl(bn_var)

    params = (att_W1, r1(att_b1), r1(att_a1), att_W2, r1(att_b2),
              r1(att_a2), att_Wf.reshape(1, 1, 40), r1(att_bf),
              *g4, *be4, *mu4, *va4,
              ffn_W1[o_u:o_i], ffn_W1[o_i:o_d], ffn_W1[o_d:o_s],
              ffn_W1[o_s:o_e].reshape(OTHER, ED, 80), r1(ffn_b1),
              r1(ffn_a1), ffn_W2, r1(ffn_b2), r1(ffn_a2),
              out_W.reshape(1, 40), r1(out_b))
    return _tc_dense(256, b0, b1, seq0, dense_inputs, sp_e, params)
